# baseline (speedup 1.0000x reference)
import functools

import jax
import jax.numpy as jnp
from jax import lax
from jax.experimental import pallas as pl
from jax.experimental.pallas import tpu as pltpu
from jax.experimental.pallas import tpu_sc as plsc

_NG = 256
_N = 10000
_E = 320000
_NC = 2
_NS = 16
_NW = _NC * _NS
_EPW = _E // _NW
_K = 80
_NCHUNK = _EPW // _K
_NPAD = 10240
_RPT = _NPAD // _NS
_ZR = 128


def _mp_layer(h, src, dst, ea, we, be, add_be):
  n, f = h.shape
  nf16 = f // 16
  mesh = plsc.VectorSubcoreMesh(core_axis_name="c", subcore_axis_name="s")

  @functools.partial(
      pl.kernel,
      out_type=jax.ShapeDtypeStruct((_NC, _NPAD, f), jnp.float32),
      mesh=mesh,
      compiler_params=pltpu.CompilerParams(use_tc_tiling_on_sc=False),
      scratch_types=(
          [pltpu.VMEM((_K,), jnp.int32)] * 2
          + [pltpu.VMEM((_K,), jnp.int32)] * 2
          + [pltpu.VMEM((_K,), jnp.float32)] * 2
          + [pltpu.VMEM((_K, f), jnp.float32)] * 2
          + [
              pltpu.VMEM((f,), jnp.float32),
              pltpu.VMEM((f,), jnp.float32),
              pltpu.VMEM((_ZR, f), jnp.float32),
              pltpu.VMEM_SHARED((_NPAD, f), jnp.float32),
          ]
          + [pltpu.SemaphoreType.DMA] * 4
      ),
  )
  def mp(h_hbm, src_hbm, dst_hbm, ea_hbm, we_hbm, be_hbm, out_hbm, *s):
    srcvs, dstvs, eavs, rowss = s[0:2], s[2:4], s[4:6], s[6:8]
    wev, bev, zbuf, agg = s[8:12]
    esems, gsems = s[12:14], s[14:16]
    cid = lax.axis_index("c")
    sid = lax.axis_index("s")
    wid = cid * _NS + sid

    zero16 = jnp.zeros((16,), jnp.float32)

    def zrow(i, carry):
      for ff in range(nf16):
        zbuf[i, pl.ds(16 * ff, 16)] = zero16
      return carry

    lax.fori_loop(0, _ZR, zrow, 0)
    row0 = sid * _RPT
    for r in range(_RPT // _ZR):
      pltpu.sync_copy(zbuf, agg.at[pl.ds(row0 + r * _ZR, _ZR)])
    pltpu.sync_copy(we_hbm, wev)
    pltpu.sync_copy(be_hbm, bev)
    plsc.subcore_barrier()

    wvals = [wev[pl.ds(16 * ff, 16)] for ff in range(nf16)]
    bvals = [bev[pl.ds(16 * ff, 16)] for ff in range(nf16)]
    ebase = wid * _EPW

    def start_edges(j, b):
      off = ebase + j * _K
      pltpu.async_copy(src_hbm.at[pl.ds(off, _K)], srcvs[b], esems[b])
      pltpu.async_copy(dst_hbm.at[pl.ds(off, _K)], dstvs[b], esems[b])
      pltpu.async_copy(ea_hbm.at[pl.ds(off, _K)], eavs[b], esems[b])

    def wait_edges(b):
      pltpu.make_async_copy(src_hbm.at[pl.ds(0, _K)], srcvs[b], esems[b]).wait()
      pltpu.make_async_copy(dst_hbm.at[pl.ds(0, _K)], dstvs[b], esems[b]).wait()
      pltpu.make_async_copy(ea_hbm.at[pl.ds(0, _K)], eavs[b], esems[b]).wait()

    def start_gather(b):
      pltpu.async_copy(h_hbm.at[srcvs[b]], rowss[b], gsems[b])

    def wait_gather(b):
      pltpu.make_async_copy(h_hbm.at[pl.ds(0, _K)], rowss[b], gsems[b]).wait()

    def scatter(b):
      pltpu.sync_copy(rowss[b], agg.at[dstvs[b]], add=True)

    def compute(b):
      eav, rows = eavs[b], rowss[b]

      def edge_grp(t, c2):
        avec = eav[pl.ds(16 * t, 16)]
        for q in range(16):
          a = avec[q]
          e = 16 * t + q
          for ff in range(nf16):
            v = rows[e, pl.ds(16 * ff, 16)]
            if add_be:
              rows[e, pl.ds(16 * ff, 16)] = jnp.maximum(
                  v + (a * wvals[ff] + bvals[ff]), 0.0)
            else:
              rows[e, pl.ds(16 * ff, 16)] = jnp.maximum(
                  v + a * wvals[ff], 0.0)
        return c2

      lax.fori_loop(0, _K // 16, edge_grp, 0, unroll=True)

    def step(j, b):
      nb = 1 - b

      @pl.when(j + 1 < _NCHUNK)
      def _():
        wait_edges(nb)
        start_gather(nb)

      wait_gather(b)
      compute(b)
      scatter(b)

      @pl.when(j + 2 < _NCHUNK)
      def _():
        start_edges(j + 2, b)

    start_edges(0, 0)
    start_edges(1, 1)
    wait_edges(0)
    start_gather(0)

    def pair(i, carry):
      step(2 * i, 0)
      step(2 * i + 1, 1)
      return carry

    lax.fori_loop(0, _NCHUNK // 2, pair, 0)
    if _NCHUNK % 2 == 1:
      wait_gather(0)
      compute(0)
      scatter(0)

    plsc.subcore_barrier()
    pltpu.sync_copy(agg.at[pl.ds(row0, _RPT)],
                    out_hbm.at[cid, pl.ds(row0, _RPT)])

  return mp(h, src, dst, ea, we, be)


def _dense_bn(h, parts, e_scale, w, b, g, bt, sub=None, extra=None):
  n, fin = h.shape
  fout = w.shape[1]
  has_sub = sub is not None
  has_extra = extra is not None

  def body(*refs):
    it = iter(refs)
    h_ref, p_ref, e_ref, w_ref, b_ref, g_ref, bt_ref = (next(it) for _ in range(7))
    sub_ref = next(it) if has_sub else None
    extra_ref = next(it) if has_extra else None
    o_ref = next(it)
    hv = h_ref[...]
    if has_sub:
      hv = hv - sub_ref[...]
    v = e_ref[...] * hv + p_ref[0, :n] + p_ref[1, :n]
    z = jnp.dot(v, w_ref[...], preferred_element_type=jnp.float32) + b_ref[...]
    mu = jnp.mean(z, axis=0, keepdims=True)
    var = jnp.mean((z - mu) ** 2, axis=0, keepdims=True)
    zn = g_ref[...] * (z - mu) * lax.rsqrt(var + 1e-5) + bt_ref[...]
    out = jnp.maximum(zn, 0.0)
    if has_extra:
      out = out + extra_ref[...]
    o_ref[...] = out

  args = [h, parts, e_scale, w, b, g, bt]
  if has_sub:
    args.append(sub)
  if has_extra:
    args.append(extra)
  return pl.pallas_call(
      body,
      out_shape=jax.ShapeDtypeStruct((n, fout), jnp.float32),
  )(*args)


def _tail(h2, parts, e_scale, sub, w3, b3, bid, states, ws, bs, wp, bp,
          wq1, bq1, wq2, bq2):
  n = h2.shape[0]
  nblk = 8
  bs_n = n // nblk

  def body(h_ref, p_ref, e_ref, sub_ref, w3_ref, b3_ref, bid_ref, st_ref,
           ws_ref, bs_ref, wp_ref, bp_ref, wq1_ref, bq1_ref,
           wq2_ref, bq2_ref, o_ref):
    v = e_ref[...] * (h_ref[...] - sub_ref[...]) + p_ref[0, :n] + p_ref[1, :n]
    h3 = jnp.maximum(
        jnp.dot(v, w3_ref[...], preferred_element_type=jnp.float32)
        + b3_ref[...], 0.0)

    giota = lax.broadcasted_iota(jnp.int32, (_NG, bs_n), 0)
    gsum = jnp.zeros((_NG, 8), jnp.float32)
    cnt = jnp.zeros((_NG, 1), jnp.float32)
    gmax = jnp.full((_NG, 8), -jnp.inf, jnp.float32)
    for t in range(nblk):
      bid_blk = bid_ref[:, t * bs_n:(t + 1) * bs_n]
      m = (giota == bid_blk).astype(jnp.float32)
      h3_blk = lax.slice(h3, (t * bs_n, 0), ((t + 1) * bs_n, 8))
      gsum = gsum + jnp.dot(m, h3_blk, preferred_element_type=jnp.float32)
      cnt = cnt + jnp.sum(m, axis=1, keepdims=True)
      h3t = jnp.transpose(h3_blk)
      fmax = []
      for ff in range(8):
        x = jnp.where(m > 0.0, h3t[ff:ff + 1, :], -jnp.inf)
        fmax.append(jnp.max(x, axis=1, keepdims=True))
      gmax = jnp.maximum(gmax, jnp.concatenate(fmax, axis=1))
    gmean = gsum / jnp.maximum(cnt, 1.0)

    sx = jnp.maximum(
        jnp.dot(st_ref[...], ws_ref[...], preferred_element_type=jnp.float32)
        + bs_ref[...], 0.0)
    inter = jnp.dot(jnp.concatenate([sx, gmax, gmean], axis=1), wp_ref[...],
                    preferred_element_type=jnp.float32) + bp_ref[...]
    q1 = jnp.dot(inter, wq1_ref[...],
                 preferred_element_type=jnp.float32) + bq1_ref[...]
    o_ref[...] = jnp.dot(q1, wq2_ref[...],
                         preferred_element_type=jnp.float32) + bq2_ref[...]

  return pl.pallas_call(
      body,
      out_shape=jax.ShapeDtypeStruct((_NG, 10), jnp.float32),
  )(h2, parts, e_scale, sub, w3, b3, bid, states, ws, bs, wp, bp,
    wq1, bq1, wq2, bq2)


def kernel(x, edge_index, edge_attr, batch_ids, states,
           eps1, We1, be1, W1, b1, g1, bt1,
           eps2, We2, be2, W2, b2, g2, bt2,
           eps3, We3, be3, W3, b3,
           Ws, bs, Wp, bp, Wq1, bq1, Wq2, bq2):
  src = edge_index[0]
  dst = edge_index[1]
  ea = edge_attr[:, 0]

  agg1 = _mp_layer(x, src, dst, ea, We1.reshape(-1), be1, True)
  h1b = _dense_bn(x, agg1, (1.0 + eps1).reshape(1, 1), W1,
                  b1.reshape(1, -1), g1.reshape(1, -1), bt1.reshape(1, -1),
                  sub=None, extra=be2.reshape(1, -1))
  agg2 = _mp_layer(h1b, src, dst, ea, We2.reshape(-1), be2, False)
  h2b = _dense_bn(h1b, agg2, (1.0 + eps2).reshape(1, 1), W2,
                  b2.reshape(1, -1), g2.reshape(1, -1), bt2.reshape(1, -1),
                  sub=be2.reshape(1, -1), extra=be3.reshape(1, -1))
  agg3 = _mp_layer(h2b, src, dst, ea, We3.reshape(-1), be3, False)
  policy = _tail(h2b, agg3, (1.0 + eps3).reshape(1, 1), be3.reshape(1, -1),
                 W3, b3.reshape(1, -1), batch_ids.reshape(1, -1), states,
                 Ws, bs.reshape(1, -1), Wp, bp.reshape(1, -1),
                 Wq1, bq1.reshape(1, -1), Wq2, bq2.reshape(1, -1))
  return policy


# speedup vs baseline: 1.2679x; 1.2679x over previous
import functools

import jax
import jax.numpy as jnp
from jax import lax
from jax.experimental import pallas as pl
from jax.experimental.pallas import tpu as pltpu
from jax.experimental.pallas import tpu_sc as plsc

_NG = 256
_N = 10000
_E = 320000
_NC = 2
_NS = 16
_NW = _NC * _NS
_EPW = _E // _NW
_K = 80
_NCHUNK = _EPW // _K
_NPAD = 10240
_RPT = _NPAD // _NS
_ZR = 128


def _mp_layer(h, src, dst, ea, we, be, add_be):
  n, f = h.shape
  nf16 = f // 16
  mesh = plsc.VectorSubcoreMesh(core_axis_name="c", subcore_axis_name="s")

  @functools.partial(
      pl.kernel,
      out_type=jax.ShapeDtypeStruct((_NC, _NPAD, f), jnp.float32),
      mesh=mesh,
      compiler_params=pltpu.CompilerParams(use_tc_tiling_on_sc=False),
      scratch_types=(
          [pltpu.VMEM((_K,), jnp.int32)] * 2
          + [pltpu.VMEM((_K,), jnp.int32)] * 2
          + [pltpu.VMEM((_K,), jnp.float32)] * 2
          + [pltpu.VMEM((_K, f), jnp.float32)] * 2
          + [
              pltpu.VMEM((f,), jnp.float32),
              pltpu.VMEM((f,), jnp.float32),
              pltpu.VMEM((_ZR, f), jnp.float32),
              pltpu.VMEM_SHARED((_NPAD, f), jnp.float32),
          ]
          + [pltpu.SemaphoreType.DMA] * 4
      ),
  )
  def mp(h_hbm, src_hbm, dst_hbm, ea_hbm, we_hbm, be_hbm, out_hbm, *s):
    srcvs, dstvs, eavs, rowss = s[0:2], s[2:4], s[4:6], s[6:8]
    wev, bev, zbuf, agg = s[8:12]
    esems, gsems = s[12:14], s[14:16]
    cid = lax.axis_index("c")
    sid = lax.axis_index("s")
    wid = cid * _NS + sid

    zero16 = jnp.zeros((16,), jnp.float32)

    def zrow(i, carry):
      for ff in range(nf16):
        zbuf[i, pl.ds(16 * ff, 16)] = zero16
      return carry

    lax.fori_loop(0, _ZR, zrow, 0)
    row0 = sid * _RPT
    for r in range(_RPT // _ZR):
      pltpu.sync_copy(zbuf, agg.at[pl.ds(row0 + r * _ZR, _ZR)])
    pltpu.sync_copy(we_hbm, wev)
    pltpu.sync_copy(be_hbm, bev)
    plsc.subcore_barrier()

    wvals = [wev[pl.ds(16 * ff, 16)] for ff in range(nf16)]
    bvals = [bev[pl.ds(16 * ff, 16)] for ff in range(nf16)]
    ebase = wid * _EPW

    def start_edges(j, b):
      off = ebase + j * _K
      pltpu.async_copy(src_hbm.at[pl.ds(off, _K)], srcvs[b], esems[b])
      pltpu.async_copy(dst_hbm.at[pl.ds(off, _K)], dstvs[b], esems[b])
      pltpu.async_copy(ea_hbm.at[pl.ds(off, _K)], eavs[b], esems[b])

    def wait_edges(b):
      pltpu.make_async_copy(src_hbm.at[pl.ds(0, _K)], srcvs[b], esems[b]).wait()
      pltpu.make_async_copy(dst_hbm.at[pl.ds(0, _K)], dstvs[b], esems[b]).wait()
      pltpu.make_async_copy(ea_hbm.at[pl.ds(0, _K)], eavs[b], esems[b]).wait()

    def start_gather(b):
      pltpu.async_copy(h_hbm.at[srcvs[b]], rowss[b], gsems[b])

    def wait_gather(b):
      pltpu.make_async_copy(h_hbm.at[pl.ds(0, _K)], rowss[b], gsems[b]).wait()

    def scatter(b):
      pltpu.sync_copy(rowss[b], agg.at[dstvs[b]], add=True)

    def compute(b):
      eav, rows = eavs[b], rowss[b]

      def edge_grp(t, c2):
        avec = eav[pl.ds(16 * t, 16)]
        for q in range(16):
          aq = lax.gather(
              avec, jnp.full((16, 1), q, jnp.int32),
              lax.GatherDimensionNumbers(offset_dims=(),
                                         collapsed_slice_dims=(0,),
                                         start_index_map=(0,)),
              (1,), mode=lax.GatherScatterMode.PROMISE_IN_BOUNDS)
          e = 16 * t + q
          for ff in range(nf16):
            v = rows[e, pl.ds(16 * ff, 16)]
            if add_be:
              rows[e, pl.ds(16 * ff, 16)] = jnp.maximum(
                  v + (aq * wvals[ff] + bvals[ff]), 0.0)
            else:
              rows[e, pl.ds(16 * ff, 16)] = jnp.maximum(
                  v + aq * wvals[ff], 0.0)
        return c2

      lax.fori_loop(0, _K // 16, edge_grp, 0)

    def step(j, b):
      nb = 1 - b

      @pl.when(j + 1 < _NCHUNK)
      def _():
        wait_edges(nb)
        start_gather(nb)

      wait_gather(b)
      compute(b)
      scatter(b)

      @pl.when(j + 2 < _NCHUNK)
      def _():
        start_edges(j + 2, b)

    start_edges(0, 0)
    start_edges(1, 1)
    wait_edges(0)
    start_gather(0)

    def pair(i, carry):
      step(2 * i, 0)
      step(2 * i + 1, 1)
      return carry

    lax.fori_loop(0, _NCHUNK // 2, pair, 0)
    if _NCHUNK % 2 == 1:
      wait_gather(0)
      compute(0)
      scatter(0)

    plsc.subcore_barrier()
    pltpu.sync_copy(agg.at[pl.ds(row0, _RPT)],
                    out_hbm.at[cid, pl.ds(row0, _RPT)])

  return mp(h, src, dst, ea, we, be)


def _dense_bn(h, parts, e_scale, w, b, g, bt, sub=None, extra=None):
  n, fin = h.shape
  fout = w.shape[1]
  has_sub = sub is not None
  has_extra = extra is not None

  def body(*refs):
    it = iter(refs)
    h_ref, p_ref, e_ref, w_ref, b_ref, g_ref, bt_ref = (next(it) for _ in range(7))
    sub_ref = next(it) if has_sub else None
    extra_ref = next(it) if has_extra else None
    o_ref = next(it)
    hv = h_ref[...]
    if has_sub:
      hv = hv - sub_ref[...]
    v = e_ref[...] * hv + p_ref[0, :n] + p_ref[1, :n]
    z = jnp.dot(v, w_ref[...], preferred_element_type=jnp.float32) + b_ref[...]
    mu = jnp.mean(z, axis=0, keepdims=True)
    var = jnp.mean((z - mu) ** 2, axis=0, keepdims=True)
    zn = g_ref[...] * (z - mu) * lax.rsqrt(var + 1e-5) + bt_ref[...]
    out = jnp.maximum(zn, 0.0)
    if has_extra:
      out = out + extra_ref[...]
    o_ref[...] = out

  args = [h, parts, e_scale, w, b, g, bt]
  if has_sub:
    args.append(sub)
  if has_extra:
    args.append(extra)
  return pl.pallas_call(
      body,
      out_shape=jax.ShapeDtypeStruct((n, fout), jnp.float32),
  )(*args)


def _tail(h2, parts, e_scale, sub, w3, b3, bid, states, ws, bs, wp, bp,
          wq1, bq1, wq2, bq2):
  n = h2.shape[0]
  nblk = 8
  bs_n = n // nblk

  def body(h_ref, p_ref, e_ref, sub_ref, w3_ref, b3_ref, bid_ref, st_ref,
           ws_ref, bs_ref, wp_ref, bp_ref, wq1_ref, bq1_ref,
           wq2_ref, bq2_ref, o_ref):
    v = e_ref[...] * (h_ref[...] - sub_ref[...]) + p_ref[0, :n] + p_ref[1, :n]
    h3 = jnp.maximum(
        jnp.dot(v, w3_ref[...], preferred_element_type=jnp.float32)
        + b3_ref[...], 0.0)

    giota = lax.broadcasted_iota(jnp.int32, (_NG, bs_n), 0)
    gsum = jnp.zeros((_NG, 8), jnp.float32)
    cnt = jnp.zeros((_NG, 1), jnp.float32)
    gmax = jnp.full((_NG, 8), -jnp.inf, jnp.float32)
    for t in range(nblk):
      bid_blk = bid_ref[:, t * bs_n:(t + 1) * bs_n]
      m = (giota == bid_blk).astype(jnp.float32)
      h3_blk = lax.slice(h3, (t * bs_n, 0), ((t + 1) * bs_n, 8))
      gsum = gsum + jnp.dot(m, h3_blk, preferred_element_type=jnp.float32)
      cnt = cnt + jnp.sum(m, axis=1, keepdims=True)
      h3t = jnp.transpose(h3_blk)
      fmax = []
      for ff in range(8):
        x = jnp.where(m > 0.0, h3t[ff:ff + 1, :], -jnp.inf)
        fmax.append(jnp.max(x, axis=1, keepdims=True))
      gmax = jnp.maximum(gmax, jnp.concatenate(fmax, axis=1))
    gmean = gsum / jnp.maximum(cnt, 1.0)

    sx = jnp.maximum(
        jnp.dot(st_ref[...], ws_ref[...], preferred_element_type=jnp.float32)
        + bs_ref[...], 0.0)
    inter = jnp.dot(jnp.concatenate([sx, gmax, gmean], axis=1), wp_ref[...],
                    preferred_element_type=jnp.float32) + bp_ref[...]
    q1 = jnp.dot(inter, wq1_ref[...],
                 preferred_element_type=jnp.float32) + bq1_ref[...]
    o_ref[...] = jnp.dot(q1, wq2_ref[...],
                         preferred_element_type=jnp.float32) + bq2_ref[...]

  return pl.pallas_call(
      body,
      out_shape=jax.ShapeDtypeStruct((_NG, 10), jnp.float32),
  )(h2, parts, e_scale, sub, w3, b3, bid, states, ws, bs, wp, bp,
    wq1, bq1, wq2, bq2)


def kernel(x, edge_index, edge_attr, batch_ids, states,
           eps1, We1, be1, W1, b1, g1, bt1,
           eps2, We2, be2, W2, b2, g2, bt2,
           eps3, We3, be3, W3, b3,
           Ws, bs, Wp, bp, Wq1, bq1, Wq2, bq2):
  src = edge_index[0]
  dst = edge_index[1]
  ea = edge_attr[:, 0]

  agg1 = _mp_layer(x, src, dst, ea, We1.reshape(-1), be1, True)
  h1b = _dense_bn(x, agg1, (1.0 + eps1).reshape(1, 1), W1,
                  b1.reshape(1, -1), g1.reshape(1, -1), bt1.reshape(1, -1),
                  sub=None, extra=be2.reshape(1, -1))
  agg2 = _mp_layer(h1b, src, dst, ea, We2.reshape(-1), be2, False)
  h2b = _dense_bn(h1b, agg2, (1.0 + eps2).reshape(1, 1), W2,
                  b2.reshape(1, -1), g2.reshape(1, -1), bt2.reshape(1, -1),
                  sub=be2.reshape(1, -1), extra=be3.reshape(1, -1))
  agg3 = _mp_layer(h2b, src, dst, ea, We3.reshape(-1), be3, False)
  policy = _tail(h2b, agg3, (1.0 + eps3).reshape(1, 1), be3.reshape(1, -1),
                 W3, b3.reshape(1, -1), batch_ids.reshape(1, -1), states,
                 Ws, bs.reshape(1, -1), Wp, bp.reshape(1, -1),
                 Wq1, bq1.reshape(1, -1), Wq2, bq2.reshape(1, -1))
  return policy


# speedup vs baseline: 1.5492x; 1.2218x over previous
import functools

import jax
import jax.numpy as jnp
from jax import lax
from jax.experimental import pallas as pl
from jax.experimental.pallas import tpu as pltpu
from jax.experimental.pallas import tpu_sc as plsc

_NG = 256
_N = 10000
_E = 320000
_NC = 2
_NS = 16
_NW = _NC * _NS
_EPW = _E // _NW
_K = 80
_NCHUNK = _EPW // _K
_NPAD = 10240
_RPT = _NPAD // _NS
_ZR = 128


def _mp_layer(h, src2, dst2, ea, we, be, add_be):
  n, f = h.shape
  nf16 = f // 16
  k = 80 if f > 16 else 400
  tpc = k // 80
  nchunk = _EPW // k
  wrows = _EPW // 80
  mesh = plsc.VectorSubcoreMesh(core_axis_name="c", subcore_axis_name="s")

  @functools.partial(
      pl.kernel,
      out_type=jax.ShapeDtypeStruct((_NC, _NPAD, f), jnp.float32),
      mesh=mesh,
      compiler_params=pltpu.CompilerParams(use_tc_tiling_on_sc=False),
      scratch_types=(
          [pltpu.VMEM((tpc, 80), jnp.int32)] * 2
          + [pltpu.VMEM((tpc, 80), jnp.int32)] * 2
          + [pltpu.VMEM((k,), jnp.float32)] * 2
          + [pltpu.VMEM((k, f), jnp.float32)] * 2
          + [
              pltpu.VMEM((f,), jnp.float32),
              pltpu.VMEM((f,), jnp.float32),
              pltpu.VMEM((_ZR, f), jnp.float32),
              pltpu.VMEM_SHARED((_NPAD, f), jnp.float32),
          ]
          + [pltpu.SemaphoreType.DMA] * 4
      ),
  )
  def mp(h_hbm, src_hbm, dst_hbm, ea_hbm, we_hbm, be_hbm, out_hbm, *s):
    srcvs, dstvs, eavs, rowss = s[0:2], s[2:4], s[4:6], s[6:8]
    wev, bev, zbuf, agg = s[8:12]
    esems, gsems = s[12:14], s[14:16]
    cid = lax.axis_index("c")
    sid = lax.axis_index("s")
    wid = cid * _NS + sid

    zero16 = jnp.zeros((16,), jnp.float32)

    def zrow(i, carry):
      for ff in range(nf16):
        zbuf[i, pl.ds(16 * ff, 16)] = zero16
      return carry

    lax.fori_loop(0, _ZR, zrow, 0)
    row0 = sid * _RPT
    for r in range(_RPT // _ZR):
      pltpu.sync_copy(zbuf, agg.at[pl.ds(row0 + r * _ZR, _ZR)])
    pltpu.sync_copy(we_hbm, wev)
    pltpu.sync_copy(be_hbm, bev)
    plsc.subcore_barrier()

    wvals = [wev[pl.ds(16 * ff, 16)] for ff in range(nf16)]
    bvals = [bev[pl.ds(16 * ff, 16)] for ff in range(nf16)]
    ebase = wid * _EPW

    def start_edges(j, b):
      row = wid * wrows + j * tpc
      pltpu.async_copy(src_hbm.at[pl.ds(row, tpc)], srcvs[b], esems[b])
      pltpu.async_copy(dst_hbm.at[pl.ds(row, tpc)], dstvs[b], esems[b])
      pltpu.async_copy(ea_hbm.at[pl.ds(ebase + j * k, k)], eavs[b], esems[b])

    def wait_edges(b):
      pltpu.make_async_copy(src_hbm.at[pl.ds(0, tpc)], srcvs[b], esems[b]).wait()
      pltpu.make_async_copy(dst_hbm.at[pl.ds(0, tpc)], dstvs[b], esems[b]).wait()
      pltpu.make_async_copy(ea_hbm.at[pl.ds(0, k)], eavs[b], esems[b]).wait()

    def start_gather(b):
      for r in range(tpc):
        pltpu.async_copy(h_hbm.at[srcvs[b].at[r]],
                         rowss[b].at[pl.ds(80 * r, 80)], gsems[b])

    def wait_gather(b):
      for r in range(tpc):
        pltpu.make_async_copy(h_hbm.at[pl.ds(0, 80)],
                              rowss[b].at[pl.ds(80 * r, 80)],
                              gsems[b]).wait()

    def scatter(b):
      for r in range(tpc):
        pltpu.sync_copy(rowss[b].at[pl.ds(80 * r, 80)],
                        agg.at[dstvs[b].at[r]], add=True)

    def compute(b):
      eav, rows = eavs[b], rowss[b]

      def edge_grp(t, c2):
        avec = eav[pl.ds(16 * t, 16)]
        for q in range(16):
          aq = lax.gather(
              avec, jnp.full((16, 1), q, jnp.int32),
              lax.GatherDimensionNumbers(offset_dims=(),
                                         collapsed_slice_dims=(0,),
                                         start_index_map=(0,)),
              (1,), mode=lax.GatherScatterMode.PROMISE_IN_BOUNDS)
          e = 16 * t + q
          for ff in range(nf16):
            v = rows[e, pl.ds(16 * ff, 16)]
            if add_be:
              rows[e, pl.ds(16 * ff, 16)] = jnp.maximum(
                  v + (aq * wvals[ff] + bvals[ff]), 0.0)
            else:
              rows[e, pl.ds(16 * ff, 16)] = jnp.maximum(
                  v + aq * wvals[ff], 0.0)
        return c2

      lax.fori_loop(0, k // 16, edge_grp, 0)

    def step(j, b):
      nb = 1 - b

      @pl.when(j + 1 < nchunk)
      def _():
        wait_edges(nb)
        start_gather(nb)

      wait_gather(b)
      compute(b)
      scatter(b)

      @pl.when(j + 2 < nchunk)
      def _():
        start_edges(j + 2, b)

    start_edges(0, 0)
    start_edges(1, 1)
    wait_edges(0)
    start_gather(0)

    def pair(i, carry):
      step(2 * i, 0)
      step(2 * i + 1, 1)
      return carry

    lax.fori_loop(0, nchunk // 2, pair, 0)
    if nchunk % 2 == 1:
      wait_gather(0)
      compute(0)
      scatter(0)

    plsc.subcore_barrier()
    pltpu.sync_copy(agg.at[pl.ds(row0, _RPT)],
                    out_hbm.at[cid, pl.ds(row0, _RPT)])

  return mp(h, src2, dst2, ea, we, be)


def _dense_bn(h, parts, e_scale, w, b, g, bt, sub=None, extra=None):
  n, fin = h.shape
  fout = w.shape[1]
  has_sub = sub is not None
  has_extra = extra is not None

  def body(*refs):
    it = iter(refs)
    h_ref, p_ref, e_ref, w_ref, b_ref, g_ref, bt_ref = (next(it) for _ in range(7))
    sub_ref = next(it) if has_sub else None
    extra_ref = next(it) if has_extra else None
    o_ref = next(it)
    hv = h_ref[...]
    if has_sub:
      hv = hv - sub_ref[...]
    v = e_ref[...] * hv + p_ref[0, :n] + p_ref[1, :n]
    z = jnp.dot(v, w_ref[...], preferred_element_type=jnp.float32) + b_ref[...]
    mu = jnp.mean(z, axis=0, keepdims=True)
    var = jnp.mean((z - mu) ** 2, axis=0, keepdims=True)
    zn = g_ref[...] * (z - mu) * lax.rsqrt(var + 1e-5) + bt_ref[...]
    out = jnp.maximum(zn, 0.0)
    if has_extra:
      out = out + extra_ref[...]
    o_ref[...] = out

  args = [h, parts, e_scale, w, b, g, bt]
  if has_sub:
    args.append(sub)
  if has_extra:
    args.append(extra)
  return pl.pallas_call(
      body,
      out_shape=jax.ShapeDtypeStruct((n, fout), jnp.float32),
  )(*args)


def _tail(h2, parts, e_scale, sub, w3, b3, bid, states, ws, bs, wp, bp,
          wq1, bq1, wq2, bq2):
  n = h2.shape[0]
  nblk = 8
  bs_n = n // nblk

  def body(h_ref, p_ref, e_ref, sub_ref, w3_ref, b3_ref, bid_ref, st_ref,
           ws_ref, bs_ref, wp_ref, bp_ref, wq1_ref, bq1_ref,
           wq2_ref, bq2_ref, o_ref):
    v = e_ref[...] * (h_ref[...] - sub_ref[...]) + p_ref[0, :n] + p_ref[1, :n]
    h3 = jnp.maximum(
        jnp.dot(v, w3_ref[...], preferred_element_type=jnp.float32)
        + b3_ref[...], 0.0)

    giota = lax.broadcasted_iota(jnp.int32, (_NG, bs_n), 0)
    gsum = jnp.zeros((_NG, 8), jnp.float32)
    cnt = jnp.zeros((_NG, 1), jnp.float32)
    gmax = jnp.full((_NG, 8), -jnp.inf, jnp.float32)
    for t in range(nblk):
      bid_blk = bid_ref[:, t * bs_n:(t + 1) * bs_n]
      m = (giota == bid_blk).astype(jnp.float32)
      h3_blk = lax.slice(h3, (t * bs_n, 0), ((t + 1) * bs_n, 8))
      gsum = gsum + jnp.dot(m, h3_blk, preferred_element_type=jnp.float32)
      cnt = cnt + jnp.sum(m, axis=1, keepdims=True)
      h3t = jnp.transpose(h3_blk)
      fmax = []
      for ff in range(8):
        x = jnp.where(m > 0.0, h3t[ff:ff + 1, :], -jnp.inf)
        fmax.append(jnp.max(x, axis=1, keepdims=True))
      gmax = jnp.maximum(gmax, jnp.concatenate(fmax, axis=1))
    gmean = gsum / jnp.maximum(cnt, 1.0)

    sx = jnp.maximum(
        jnp.dot(st_ref[...], ws_ref[...], preferred_element_type=jnp.float32)
        + bs_ref[...], 0.0)
    inter = jnp.dot(jnp.concatenate([sx, gmax, gmean], axis=1), wp_ref[...],
                    preferred_element_type=jnp.float32) + bp_ref[...]
    q1 = jnp.dot(inter, wq1_ref[...],
                 preferred_element_type=jnp.float32) + bq1_ref[...]
    o_ref[...] = jnp.dot(q1, wq2_ref[...],
                         preferred_element_type=jnp.float32) + bq2_ref[...]

  return pl.pallas_call(
      body,
      out_shape=jax.ShapeDtypeStruct((_NG, 10), jnp.float32),
  )(h2, parts, e_scale, sub, w3, b3, bid, states, ws, bs, wp, bp,
    wq1, bq1, wq2, bq2)


def kernel(x, edge_index, edge_attr, batch_ids, states,
           eps1, We1, be1, W1, b1, g1, bt1,
           eps2, We2, be2, W2, b2, g2, bt2,
           eps3, We3, be3, W3, b3,
           Ws, bs, Wp, bp, Wq1, bq1, Wq2, bq2):
  src2 = edge_index[0].reshape(-1, 80)
  dst2 = edge_index[1].reshape(-1, 80)
  ea = edge_attr[:, 0]

  agg1 = _mp_layer(x, src2, dst2, ea, We1.reshape(-1), be1, True)
  h1b = _dense_bn(x, agg1, (1.0 + eps1).reshape(1, 1), W1,
                  b1.reshape(1, -1), g1.reshape(1, -1), bt1.reshape(1, -1),
                  sub=None, extra=be2.reshape(1, -1))
  agg2 = _mp_layer(h1b, src2, dst2, ea, We2.reshape(-1), be2, False)
  h2b = _dense_bn(h1b, agg2, (1.0 + eps2).reshape(1, 1), W2,
                  b2.reshape(1, -1), g2.reshape(1, -1), bt2.reshape(1, -1),
                  sub=be2.reshape(1, -1), extra=be3.reshape(1, -1))
  agg3 = _mp_layer(h2b, src2, dst2, ea, We3.reshape(-1), be3, False)
  policy = _tail(h2b, agg3, (1.0 + eps3).reshape(1, 1), be3.reshape(1, -1),
                 W3, b3.reshape(1, -1), batch_ids.reshape(1, -1), states,
                 Ws, bs.reshape(1, -1), Wp, bp.reshape(1, -1),
                 Wq1, bq1.reshape(1, -1), Wq2, bq2.reshape(1, -1))
  return policy


# speedup vs baseline: 1.5516x; 1.0016x over previous
import functools

import jax
import jax.numpy as jnp
from jax import lax
from jax.experimental import pallas as pl
from jax.experimental.pallas import tpu as pltpu
from jax.experimental.pallas import tpu_sc as plsc

_NG = 256
_N = 10000
_E = 320000
_NC = 2
_NS = 16
_NW = _NC * _NS
_EPW = _E // _NW
_K = 80
_NCHUNK = _EPW // _K
_NPAD = 10240
_RPT = _NPAD // _NS
_ZR = 80


def _mp_layer(h, src2, dst2, ea, we, be, add_be):
  n, f = h.shape
  nf16 = f // 16
  k = 80 if f > 16 else 400
  tpc = k // 80
  nchunk = _EPW // k
  wrows = _EPW // 80
  mesh = plsc.VectorSubcoreMesh(core_axis_name="c", subcore_axis_name="s")

  @functools.partial(
      pl.kernel,
      out_type=jax.ShapeDtypeStruct((_NC, _NPAD, f), jnp.float32),
      mesh=mesh,
      compiler_params=pltpu.CompilerParams(use_tc_tiling_on_sc=False),
      scratch_types=(
          [pltpu.VMEM((tpc, 80), jnp.int32)] * 2
          + [pltpu.VMEM((tpc, 80), jnp.int32)] * 2
          + [pltpu.VMEM((k,), jnp.float32)] * 2
          + [pltpu.VMEM((k, f), jnp.float32)] * 2
          + [
              pltpu.VMEM((f,), jnp.float32),
              pltpu.VMEM((f,), jnp.float32),
              pltpu.VMEM((_ZR, f), jnp.float32),
              pltpu.VMEM_SHARED((_NPAD, f), jnp.float32),
          ]
          + [pltpu.SemaphoreType.DMA] * 4
      ),
  )
  def mp(h_hbm, src_hbm, dst_hbm, ea_hbm, we_hbm, be_hbm, out_hbm, *s):
    srcvs, dstvs, eavs, rowss = s[0:2], s[2:4], s[4:6], s[6:8]
    wev, bev, zbuf, agg = s[8:12]
    esems, gsems = s[12:14], s[14:16]
    cid = lax.axis_index("c")
    sid = lax.axis_index("s")
    wid = cid * _NS + sid

    zero16 = jnp.zeros((16,), jnp.float32)

    def zrow(i, carry):
      for ff in range(nf16):
        zbuf[i, pl.ds(16 * ff, 16)] = zero16
      return carry

    lax.fori_loop(0, _ZR, zrow, 0)
    row0 = sid * _RPT
    for r in range(_RPT // _ZR):
      pltpu.sync_copy(zbuf, agg.at[pl.ds(row0 + r * _ZR, _ZR)])
    pltpu.sync_copy(we_hbm, wev)
    pltpu.sync_copy(be_hbm, bev)
    plsc.subcore_barrier()

    wvals = [wev[pl.ds(16 * ff, 16)] for ff in range(nf16)]
    bvals = [bev[pl.ds(16 * ff, 16)] for ff in range(nf16)]
    ebase = wid * _EPW

    def start_edges(j, b):
      row = wid * wrows + j * tpc
      pltpu.async_copy(src_hbm.at[pl.ds(row, tpc)], srcvs[b], esems[b])
      pltpu.async_copy(dst_hbm.at[pl.ds(row, tpc)], dstvs[b], esems[b])
      pltpu.async_copy(ea_hbm.at[pl.ds(ebase + j * k, k)], eavs[b], esems[b])

    def wait_edges(b):
      pltpu.make_async_copy(src_hbm.at[pl.ds(0, tpc)], srcvs[b], esems[b]).wait()
      pltpu.make_async_copy(dst_hbm.at[pl.ds(0, tpc)], dstvs[b], esems[b]).wait()
      pltpu.make_async_copy(ea_hbm.at[pl.ds(0, k)], eavs[b], esems[b]).wait()

    def start_gather(b):
      for r in range(tpc):
        pltpu.async_copy(h_hbm.at[srcvs[b].at[r]],
                         rowss[b].at[pl.ds(80 * r, 80)], gsems[b])

    def wait_gather(b):
      for r in range(tpc):
        pltpu.make_async_copy(h_hbm.at[pl.ds(0, 80)],
                              rowss[b].at[pl.ds(80 * r, 80)],
                              gsems[b]).wait()

    def scatter(b):
      for r in range(tpc):
        pltpu.sync_copy(rowss[b].at[pl.ds(80 * r, 80)],
                        agg.at[dstvs[b].at[r]], add=True)

    def compute(b):
      eav, rows = eavs[b], rowss[b]

      def edge_grp(t, c2):
        avec = eav[pl.ds(16 * t, 16)]
        for q in range(16):
          aq = lax.gather(
              avec, jnp.full((16, 1), q, jnp.int32),
              lax.GatherDimensionNumbers(offset_dims=(),
                                         collapsed_slice_dims=(0,),
                                         start_index_map=(0,)),
              (1,), mode=lax.GatherScatterMode.PROMISE_IN_BOUNDS)
          e = 16 * t + q
          for ff in range(nf16):
            v = rows[e, pl.ds(16 * ff, 16)]
            if add_be:
              rows[e, pl.ds(16 * ff, 16)] = jnp.maximum(
                  v + (aq * wvals[ff] + bvals[ff]), 0.0)
            else:
              rows[e, pl.ds(16 * ff, 16)] = jnp.maximum(
                  v + aq * wvals[ff], 0.0)
        return c2

      lax.fori_loop(0, k // 16, edge_grp, 0)

    def step(j, b):
      nb = 1 - b

      @pl.when(j + 1 < nchunk)
      def _():
        wait_edges(nb)
        start_gather(nb)

      wait_gather(b)
      compute(b)
      scatter(b)

      @pl.when(j + 2 < nchunk)
      def _():
        start_edges(j + 2, b)

    start_edges(0, 0)
    start_edges(1, 1)
    wait_edges(0)
    start_gather(0)

    def pair(i, carry):
      step(2 * i, 0)
      step(2 * i + 1, 1)
      return carry

    lax.fori_loop(0, nchunk // 2, pair, 0)
    if nchunk % 2 == 1:
      wait_gather(0)
      compute(0)
      scatter(0)

    plsc.subcore_barrier()
    pltpu.sync_copy(agg.at[pl.ds(row0, _RPT)],
                    out_hbm.at[cid, pl.ds(row0, _RPT)])

  return mp(h, src2, dst2, ea, we, be)


def _dense_bn(h, parts, e_scale, w, b, g, bt, sub=None, extra=None):
  n, fin = h.shape
  fout = w.shape[1]
  has_sub = sub is not None
  has_extra = extra is not None

  def body(*refs):
    it = iter(refs)
    h_ref, p_ref, e_ref, w_ref, b_ref, g_ref, bt_ref = (next(it) for _ in range(7))
    sub_ref = next(it) if has_sub else None
    extra_ref = next(it) if has_extra else None
    o_ref = next(it)
    hv = h_ref[...]
    if has_sub:
      hv = hv - sub_ref[...]
    v = e_ref[...] * hv + p_ref[0, :n] + p_ref[1, :n]
    z = jnp.dot(v, w_ref[...], preferred_element_type=jnp.float32) + b_ref[...]
    mu = jnp.mean(z, axis=0, keepdims=True)
    var = jnp.mean((z - mu) ** 2, axis=0, keepdims=True)
    zn = g_ref[...] * (z - mu) * lax.rsqrt(var + 1e-5) + bt_ref[...]
    out = jnp.maximum(zn, 0.0)
    if has_extra:
      out = out + extra_ref[...]
    o_ref[...] = out

  args = [h, parts, e_scale, w, b, g, bt]
  if has_sub:
    args.append(sub)
  if has_extra:
    args.append(extra)
  return pl.pallas_call(
      body,
      out_shape=jax.ShapeDtypeStruct((n, fout), jnp.float32),
  )(*args)


def _tail(h2, parts, e_scale, sub, w3, b3, bid, states, ws, bs, wp, bp,
          wq1, bq1, wq2, bq2):
  n = h2.shape[0]
  nblk = 8
  bs_n = n // nblk

  def body(h_ref, p_ref, e_ref, sub_ref, w3_ref, b3_ref, bid_ref, st_ref,
           ws_ref, bs_ref, wp_ref, bp_ref, wq1_ref, bq1_ref,
           wq2_ref, bq2_ref, o_ref):
    v = e_ref[...] * (h_ref[...] - sub_ref[...]) + p_ref[0, :n] + p_ref[1, :n]
    h3 = jnp.maximum(
        jnp.dot(v, w3_ref[...], preferred_element_type=jnp.float32)
        + b3_ref[...], 0.0)

    giota = lax.broadcasted_iota(jnp.int32, (_NG, bs_n), 0)
    gsum = jnp.zeros((_NG, 8), jnp.float32)
    cnt = jnp.zeros((_NG, 1), jnp.float32)
    gmax = jnp.full((_NG, 8), -jnp.inf, jnp.float32)
    for t in range(nblk):
      bid_blk = bid_ref[:, t * bs_n:(t + 1) * bs_n]
      m = (giota == bid_blk).astype(jnp.float32)
      h3_blk = lax.slice(h3, (t * bs_n, 0), ((t + 1) * bs_n, 8))
      gsum = gsum + jnp.dot(m, h3_blk, preferred_element_type=jnp.float32)
      cnt = cnt + jnp.sum(m, axis=1, keepdims=True)
      h3t = jnp.transpose(h3_blk)
      fmax = []
      for ff in range(8):
        x = jnp.where(m > 0.0, h3t[ff:ff + 1, :], -jnp.inf)
        fmax.append(jnp.max(x, axis=1, keepdims=True))
      gmax = jnp.maximum(gmax, jnp.concatenate(fmax, axis=1))
    gmean = gsum / jnp.maximum(cnt, 1.0)

    sx = jnp.maximum(
        jnp.dot(st_ref[...], ws_ref[...], preferred_element_type=jnp.float32)
        + bs_ref[...], 0.0)
    inter = jnp.dot(jnp.concatenate([sx, gmax, gmean], axis=1), wp_ref[...],
                    preferred_element_type=jnp.float32) + bp_ref[...]
    q1 = jnp.dot(inter, wq1_ref[...],
                 preferred_element_type=jnp.float32) + bq1_ref[...]
    o_ref[...] = jnp.dot(q1, wq2_ref[...],
                         preferred_element_type=jnp.float32) + bq2_ref[...]

  return pl.pallas_call(
      body,
      out_shape=jax.ShapeDtypeStruct((_NG, 10), jnp.float32),
  )(h2, parts, e_scale, sub, w3, b3, bid, states, ws, bs, wp, bp,
    wq1, bq1, wq2, bq2)


def kernel(x, edge_index, edge_attr, batch_ids, states,
           eps1, We1, be1, W1, b1, g1, bt1,
           eps2, We2, be2, W2, b2, g2, bt2,
           eps3, We3, be3, W3, b3,
           Ws, bs, Wp, bp, Wq1, bq1, Wq2, bq2):
  src2 = edge_index[0].reshape(-1, 80)
  dst2 = edge_index[1].reshape(-1, 80)
  ea = edge_attr[:, 0]

  agg1 = _mp_layer(x, src2, dst2, ea, We1.reshape(-1), be1, True)
  h1b = _dense_bn(x, agg1, (1.0 + eps1).reshape(1, 1), W1,
                  b1.reshape(1, -1), g1.reshape(1, -1), bt1.reshape(1, -1),
                  sub=None, extra=be2.reshape(1, -1))
  agg2 = _mp_layer(h1b, src2, dst2, ea, We2.reshape(-1), be2, False)
  h2b = _dense_bn(h1b, agg2, (1.0 + eps2).reshape(1, 1), W2,
                  b2.reshape(1, -1), g2.reshape(1, -1), bt2.reshape(1, -1),
                  sub=be2.reshape(1, -1), extra=be3.reshape(1, -1))
  agg3 = _mp_layer(h2b, src2, dst2, ea, We3.reshape(-1), be3, False)
  policy = _tail(h2b, agg3, (1.0 + eps3).reshape(1, 1), be3.reshape(1, -1),
                 W3, b3.reshape(1, -1), batch_ids.reshape(1, -1), states,
                 Ws, bs.reshape(1, -1), Wp, bp.reshape(1, -1),
                 Wq1, bq1.reshape(1, -1), Wq2, bq2.reshape(1, -1))
  return policy
